# baseline (speedup 1.0000x reference)
import jax
import jax.numpy as jnp
from jax import lax
from jax.experimental import pallas as pl
from jax.experimental.pallas import tpu as pltpu
from jax.experimental.pallas import tpu_sc as plsc

N_CLASSES = 1000
EMBED_DIM = 128
BATCH = 16384

_NC = 2
_NS = 16
_NW = _NC * _NS
_BPW = BATCH // _NW
_HBM_CHUNKS = [64, 64]
_SP_CHUNKS = [128, 128, 128]
assert sum(_HBM_CHUNKS) + sum(_SP_CHUNKS) == _BPW


def _gather_body(idx_hbm, table_hbm, out_hbm, idx_v, rows_v, table_sh,
                 sem, wsem):
    sid = lax.axis_index("s")
    wid = sid * _NC + lax.axis_index("c")
    base = wid * _BPW

    pltpu.sync_copy(idx_hbm.at[pl.ds(base, _BPW)], idx_v)

    gathers = []
    off = 0
    for n in _HBM_CHUNKS:
        gathers.append((off, n, pltpu.async_copy(
            table_hbm.at[idx_v.at[pl.ds(off, n)]],
            rows_v.at[pl.ds(off, n)], sem)))
        off += n

    @pl.when(sid == 0)
    def _stage():
        pltpu.sync_copy(table_hbm, table_sh)

    plsc.subcore_barrier()
    for n in _SP_CHUNKS:
        gathers.append((off, n, pltpu.async_copy(
            table_sh.at[idx_v.at[pl.ds(off, n)]],
            rows_v.at[pl.ds(off, n)], sem)))
        off += n

    writes = []
    for off, n, g in gathers:
        g.wait()
        writes.append(pltpu.async_copy(
            rows_v.at[pl.ds(off, n)], out_hbm.at[pl.ds(base + off, n)], wsem))
    for w in writes:
        w.wait()


def kernel(class_idx, table):
    idx = class_idx.astype(jnp.int32)
    mesh = plsc.VectorSubcoreMesh(core_axis_name="c", subcore_axis_name="s")
    out = pl.kernel(
        _gather_body,
        mesh=mesh,
        out_type=jax.ShapeDtypeStruct((BATCH, EMBED_DIM), jnp.float32),
        scratch_types=[
            pltpu.VMEM((_BPW,), jnp.int32),
            pltpu.VMEM((_BPW, EMBED_DIM), jnp.float32),
            pltpu.VMEM_SHARED((N_CLASSES, EMBED_DIM), jnp.float32),
            pltpu.SemaphoreType.DMA,
            pltpu.SemaphoreType.DMA,
        ],
    )(idx, table)
    return out.reshape(BATCH, 1, EMBED_DIM)


# speedup vs baseline: 1.0094x; 1.0094x over previous
import jax
import jax.numpy as jnp
from jax import lax
from jax.experimental import pallas as pl
from jax.experimental.pallas import tpu as pltpu
from jax.experimental.pallas import tpu_sc as plsc

N_CLASSES = 1000
EMBED_DIM = 128
BATCH = 16384

_NC = 2
_NS = 16
_NW = _NC * _NS
_BPW = BATCH // _NW
_CHUNK = 128
_NCHUNK = _BPW // _CHUNK


def _gather_body(idx_hbm, table_hbm, out_hbm, idx_v, rows_v, table_sh, sem, wsem):
    sid = lax.axis_index("s")
    wid = sid * _NC + lax.axis_index("c")
    base = wid * _BPW

    @pl.when(sid == 0)
    def _stage():
        pltpu.sync_copy(table_hbm, table_sh)

    idx_copy = pltpu.async_copy(idx_hbm.at[pl.ds(base, _BPW)], idx_v, sem)
    plsc.subcore_barrier()
    idx_copy.wait()
    gathers = [
        pltpu.async_copy(
            table_sh.at[idx_v.at[pl.ds(j * _CHUNK, _CHUNK)]], rows_v.at[j], sem
        )
        for j in range(_NCHUNK)
    ]
    writes = []
    for j in range(_NCHUNK):
        gathers[j].wait()
        writes.append(
            pltpu.async_copy(
                rows_v.at[j], out_hbm.at[pl.ds(base + j * _CHUNK, _CHUNK)], wsem
            )
        )
    for w in writes:
        w.wait()


def kernel(class_idx, table):
    idx = class_idx.astype(jnp.int32)
    mesh = plsc.VectorSubcoreMesh(core_axis_name="c", subcore_axis_name="s")
    out = pl.kernel(
        _gather_body,
        mesh=mesh,
        out_type=jax.ShapeDtypeStruct((BATCH, EMBED_DIM), jnp.float32),
        scratch_types=[
            pltpu.VMEM((_BPW,), jnp.int32),
            pltpu.VMEM((_NCHUNK, _CHUNK, EMBED_DIM), jnp.float32),
            pltpu.VMEM_SHARED((N_CLASSES, EMBED_DIM), jnp.float32),
            pltpu.SemaphoreType.DMA,
            pltpu.SemaphoreType.DMA,
        ],
    )(idx, table)
    return out.reshape(BATCH, 1, EMBED_DIM)
